# baseline (speedup 1.0000x reference)
import functools

import jax
import jax.numpy as jnp
from jax import lax
from jax.experimental import pallas as pl
from jax.experimental.pallas import tpu as pltpu
from jax.experimental.pallas import tpu_sc as plsc

N = 10000
E = 320000
D = 128
STEPS = 5
DT = 1.0 / STEPS
LAM = 0.1
LN_EPS = 1e-5

NC = 2
NS = 16
NW = NC * NS
CH = 128
NCHUNK = 80
ECP = NCHUNK * CH
EP = NW * ECP
EC = E // NW
CC = 2000
NP_ = 10112
RPT = NP_ // NS
BN = 1000

@functools.cache
def _mesh():
    return plsc.VectorSubcoreMesh(core_axis_name="c", subcore_axis_name="s",
                                  num_cores=NC, num_subcores=NS)


_SC_PARAMS = pltpu.CompilerParams(needs_layout_passes=False)


def _sc_prep(dst, ts):
    @functools.partial(
        pl.kernel,
        out_type=jax.ShapeDtypeStruct((NW * N,), jnp.float32),
        mesh=_mesh(),
        compiler_params=_SC_PARAMS,
        scratch_types=[pltpu.VMEM((N,), jnp.float32),
                       pltpu.VMEM((CC,), jnp.int32),
                       pltpu.VMEM((CC,), jnp.float32)])
    def k(dst_hbm, ts_hbm, degp_hbm, deg_v, dst_v, ts_v):
        wid = lax.axis_index("c") * NS + lax.axis_index("s")

        @pl.loop(0, N // 16)
        def _(i):
            deg_v[pl.ds(i * 16, 16)] = jnp.zeros((16,), jnp.float32)

        @pl.loop(0, EC // CC)
        def _(ci):
            base = wid * EC + ci * CC
            pltpu.sync_copy(dst_hbm.at[pl.ds(base, CC)], dst_v)
            pltpu.sync_copy(ts_hbm.at[pl.ds(base, CC)], ts_v)

            @pl.loop(0, CC // 16)
            def _(gi):
                wv = jnp.exp(LAM * ts_v[pl.ds(gi * 16, 16)])
                dv = dst_v[pl.ds(gi * 16, 16)]
                plsc.addupdate_scatter(deg_v, [dv], wv)

        pltpu.sync_copy(deg_v, degp_hbm.at[pl.ds(wid * N, N)])

    return k(dst, ts)


def _sc_step(edata, g, zrows):
    @functools.partial(
        pl.kernel,
        out_type=jax.ShapeDtypeStruct((NC, NP_, D), jnp.float32),
        mesh=_mesh(),
        compiler_params=_SC_PARAMS,
        scratch_types=[pltpu.VMEM((CH, D), jnp.float32),
                       pltpu.VMEM((CH, D), jnp.float32),
                       pltpu.VMEM((CH, D), jnp.float32),
                       pltpu.VMEM((3 * CH,), jnp.int32),
                       pltpu.VMEM((3 * CH,), jnp.int32),
                       pltpu.VMEM((CH,), jnp.int32),
                       pltpu.VMEM((CH,), jnp.int32),
                       pltpu.VMEM_SHARED((NP_, D), jnp.float32),
                       pltpu.SemaphoreType.DMA,
                       pltpu.SemaphoreType.DMA,
                       pltpu.SemaphoreType.DMA,
                       pltpu.SemaphoreType.DMA,
                       pltpu.SemaphoreType.DMA])
    def k(ed_hbm, g_hbm, z_hbm, part_hbm,
          rows0, rows1, scl, edv0, edv1, dsts0, dsts1, acc_sh,
          gsem0, gsem1, ssem, esem0, esem1):
        c = lax.axis_index("c")
        s = lax.axis_index("s")
        wid = c * NS + s
        ebase = wid * NCHUNK * 3 * CH
        iota = lax.iota(jnp.int32, 16)
        bufs = ((rows0, edv0, dsts0, gsem0, esem0),
                (rows1, edv1, dsts1, gsem1, esem1))

        pltpu.sync_copy(z_hbm, acc_sh.at[pl.ds(s * RPT, RPT), :])
        plsc.subcore_barrier()

        def accum(rows_b, edv_b, dsts_b):
            @pl.loop(0, CH // 16)
            def _(gi):
                dsts_b[pl.ds(gi * 16, 16)] = edv_b[pl.ds(CH + gi * 16, 16)]

            @pl.loop(0, CH // 16)
            def _(gi):
                tsv = plsc.bitcast(edv_b[pl.ds(2 * CH + gi * 16, 16)],
                                   jnp.float32)
                wv = jnp.exp(LAM * tsv)
                rowid = gi * 16 + iota
                for f in range(0, D, 2):
                    col = jnp.full((16,), f, jnp.int32)
                    col2 = jnp.full((16,), f + 1, jnp.int32)
                    v = plsc.load_gather(rows_b, [rowid, col])
                    v2 = plsc.load_gather(rows_b, [rowid, col2])
                    plsc.store_scatter(scl, [rowid, col], v * wv)
                    plsc.store_scatter(scl, [rowid, col2], v2 * wv)

        pltpu.sync_copy(ed_hbm.at[pl.ds(ebase, 3 * CH)], edv0)
        pltpu.async_copy(ed_hbm.at[pl.ds(ebase + 3 * CH, 3 * CH)], edv1,
                         esem1)
        pltpu.async_copy(g_hbm.at[edv0.at[pl.ds(0, CH)]], rows0, gsem0)

        @pl.loop(0, NCHUNK, step=2)
        def _(base):
            for b in range(2):
                ci = base + b
                rows_b, edv_b, dsts_b, gsem_b, esem_b = bufs[b]
                rows_o, edv_o, dsts_o, gsem_o, esem_o = bufs[1 - b]

                @pl.when(ci + 1 < NCHUNK)
                def _():
                    pltpu.make_async_copy(ed_hbm.at[pl.ds(0, 3 * CH)], edv_o,
                                          esem_o).wait()
                    pltpu.async_copy(g_hbm.at[edv_o.at[pl.ds(0, CH)]], rows_o,
                                     gsem_o)

                pltpu.make_async_copy(g_hbm.at[pl.ds(0, CH), :], rows_b,
                                      gsem_b).wait()

                @pl.when(ci >= 1)
                def _():
                    pltpu.make_async_copy(g_hbm.at[pl.ds(0, CH), :], scl,
                                          ssem).wait()
                accum(rows_b, edv_b, dsts_b)
                pltpu.async_copy(scl, acc_sh.at[dsts_b], ssem, add=True)

                @pl.when(ci + 2 < NCHUNK)
                def _():
                    pltpu.async_copy(
                        ed_hbm.at[pl.ds(ebase + (ci + 2) * 3 * CH, 3 * CH)],
                        edv_b, esem_b)

        pltpu.make_async_copy(g_hbm.at[pl.ds(0, CH), :], scl, ssem).wait()
        plsc.subcore_barrier()
        pltpu.sync_copy(acc_sh.at[pl.ds(s * RPT, RPT), :],
                        part_hbm.at[c, pl.ds(s * RPT, RPT), :])

    return k(edata, g, zrows)


def _tc_coeffs(degp, x, timef):
    def body(t_ref, degp_ref, x_ref, a_ref, b_ref, s_ref, g_ref):
        wl = jnp.exp(LAM * t_ref[0, 0])
        deg = jnp.sum(degp_ref[...], axis=1, keepdims=True) + wl
        inv = 1.0 / deg
        a_ref[...] = (1.0 - DT) + (DT * wl) * inv
        b_ref[...] = DT * inv
        dis = lax.rsqrt(deg)
        s_ref[...] = deg * dis
        g_ref[...] = x_ref[...] * dis

    sd = jax.ShapeDtypeStruct
    return pl.pallas_call(
        body,
        grid=(N // BN,),
        in_specs=[pl.BlockSpec(memory_space=pltpu.SMEM),
                  pl.BlockSpec((BN, NW), lambda i: (i, 0)),
                  pl.BlockSpec((BN, D), lambda i: (i, 0))],
        out_specs=[pl.BlockSpec((BN, 1), lambda i: (i, 0)),
                   pl.BlockSpec((BN, 1), lambda i: (i, 0)),
                   pl.BlockSpec((BN, 1), lambda i: (i, 0)),
                   pl.BlockSpec((BN, D), lambda i: (i, 0))],
        out_shape=(sd((N, 1), jnp.float32), sd((N, 1), jnp.float32),
                   sd((N, 1), jnp.float32), sd((N, D), jnp.float32)),
    )(timef, degp, x)


def _tc_update(g, alpha, betac, parts):
    def body(g_ref, a_ref, b_ref, p_ref, o_ref):
        o_ref[...] = (a_ref[...] * g_ref[...]
                      + b_ref[...] * (p_ref[0] + p_ref[1]))

    return pl.pallas_call(
        body,
        grid=(N // BN,),
        in_specs=[pl.BlockSpec((BN, D), lambda i: (i, 0)),
                  pl.BlockSpec((BN, 1), lambda i: (i, 0)),
                  pl.BlockSpec((BN, 1), lambda i: (i, 0)),
                  pl.BlockSpec((NC, BN, D), lambda i: (0, i, 0))],
        out_specs=pl.BlockSpec((BN, D), lambda i: (i, 0)),
        out_shape=jax.ShapeDtypeStruct((N, D), jnp.float32),
    )(g, alpha, betac, parts)


def _tc_final(g, sdeg, x, W_t, b_t, W_r, gamma, beta):
    def body(g_ref, s_ref, x_ref, wt_ref, bt_ref, wr_ref, ga_ref, be_ref,
             o_ref):
        h = g_ref[...] * s_ref[...]
        dn = (((1,), (1,)), ((), ()))
        t1 = lax.dot_general(h, wt_ref[...], dn,
                             preferred_element_type=jnp.float32,
                             precision=lax.Precision.HIGHEST)
        t1 = jnp.maximum(t1 + bt_ref[...], 0.0)
        t2 = lax.dot_general(x_ref[...], wr_ref[...], dn,
                             preferred_element_type=jnp.float32,
                             precision=lax.Precision.HIGHEST)
        o = t1 + t2
        mu = jnp.mean(o, axis=1, keepdims=True)
        d0 = o - mu
        var = jnp.mean(d0 * d0, axis=1, keepdims=True)
        o_ref[...] = d0 * lax.rsqrt(var + LN_EPS) * ga_ref[...] + be_ref[...]

    full = pl.BlockSpec((D, D), lambda i: (0, 0))
    row = pl.BlockSpec((1, D), lambda i: (0, 0))
    blk = pl.BlockSpec((BN, D), lambda i: (i, 0))
    return pl.pallas_call(
        body,
        grid=(N // BN,),
        in_specs=[blk, pl.BlockSpec((BN, 1), lambda i: (i, 0)), blk,
                  full, row, full, row, row],
        out_specs=blk,
        out_shape=jax.ShapeDtypeStruct((N, D), jnp.float32),
    )(g, sdeg, x, W_t, b_t.reshape(1, D), W_r, gamma.reshape(1, D),
      beta.reshape(1, D))


def kernel(x, edge_index, timestamps, time, W_t, b_t, W_r, gamma, beta):
    src = edge_index[0]
    dst = edge_index[1]
    timef = jnp.asarray(time, jnp.float32).reshape(1, 1)
    zrows = jnp.zeros((RPT, D), jnp.float32)
    pad = EP - E
    srcp = jnp.concatenate([src, jnp.zeros((pad,), jnp.int32)])
    dstp = jnp.concatenate([dst, jnp.full((pad,), N, jnp.int32)])
    tsp = jnp.concatenate([timestamps, jnp.zeros((pad,), jnp.float32)])
    edata = jnp.stack(
        [srcp.reshape(NW * NCHUNK, CH), dstp.reshape(NW * NCHUNK, CH),
         lax.bitcast_convert_type(tsp, jnp.int32).reshape(NW * NCHUNK, CH)],
        axis=1).reshape(-1)
    degp = _sc_prep(dst, timestamps)
    alpha, betac, sdeg, g = _tc_coeffs(degp.reshape(NW, N).T, x, timef)
    for _ in range(STEPS):
        parts = _sc_step(edata, g, zrows)
        g = _tc_update(g, alpha, betac, parts[:, :N, :])
    return _tc_final(g, sdeg, x, W_t, b_t, W_r, gamma, beta)
